# baseline (speedup 1.0000x reference)
import functools

import jax
import jax.numpy as jnp
from jax import lax
from jax.experimental import pallas as pl
from jax.experimental.pallas import tpu as pltpu
from jax.experimental.pallas import tpu_sc as plsc

E = 64
H = 1024
F = 768
TILE = 128
EPG = 2


def _sc_gather_scale(ys, idx, routing):
    T, D = ys.shape
    info = plsc.get_sparse_core_info()
    nw = info.num_cores * info.num_subcores
    b_per_w = T // nw
    mesh = plsc.VectorSubcoreMesh(core_axis_name="c", subcore_axis_name="s")

    @functools.partial(
        pl.kernel,
        out_type=jax.ShapeDtypeStruct((T, D), jnp.float32),
        mesh=mesh,
        compiler_params=pltpu.CompilerParams(needs_layout_passes=False),
        scratch_types=[
            pltpu.VMEM((b_per_w,), jnp.int32),
            pltpu.VMEM((b_per_w,), jnp.float32),
            pltpu.VMEM((b_per_w, D), jnp.float32),
            pltpu.SemaphoreType.DMA,
        ],
    )
    def k(ys_hbm, idx_hbm, r_hbm, out_hbm, idx_v, r_v, rows_v, sem):
        wid = lax.axis_index("s") * info.num_cores + lax.axis_index("c")
        base = wid * b_per_w
        pltpu.sync_copy(idx_hbm.at[pl.ds(base, b_per_w)], idx_v)
        pltpu.sync_copy(r_hbm.at[pl.ds(base, b_per_w)], r_v)
        pltpu.async_copy(ys_hbm.at[idx_v], rows_v, sem).wait()

        def row_body(i, _):
            ri = plsc.load_gather(r_v, [jnp.zeros((16,), jnp.int32) + i])
            for c in range(D // 16):
                rows_v[i, pl.ds(16 * c, 16)] = (
                    rows_v[i, pl.ds(16 * c, 16)] * ri)
            return 0

        lax.fori_loop(0, b_per_w, row_body, 0)
        pltpu.sync_copy(rows_v, out_hbm.at[pl.ds(base, b_per_w)])

    return k(ys, idx, routing)


def _sc_gather(table, idx):
    T, D = table.shape
    B = idx.shape[0]
    info = plsc.get_sparse_core_info()
    nw = info.num_cores * info.num_subcores
    b_per_w = B // nw
    mesh = plsc.VectorSubcoreMesh(core_axis_name="c", subcore_axis_name="s")

    @functools.partial(
        pl.kernel,
        out_type=jax.ShapeDtypeStruct((B, D), table.dtype),
        mesh=mesh,
        scratch_types=[
            pltpu.VMEM((b_per_w,), jnp.int32),
            pltpu.VMEM((b_per_w, D), table.dtype),
            pltpu.SemaphoreType.DMA,
        ],
    )
    def k(table_hbm, idx_hbm, out_hbm, idx_v, rows_v, sem):
        wid = lax.axis_index("s") * info.num_cores + lax.axis_index("c")
        base = wid * b_per_w
        pltpu.sync_copy(idx_hbm.at[pl.ds(base, b_per_w)], idx_v)
        pltpu.async_copy(table_hbm.at[idx_v], rows_v, sem).wait()
        pltpu.sync_copy(rows_v, out_hbm.at[pl.ds(base, b_per_w)])

    return k(table, idx)


def _sc_scatter(rows, idx, T):
    B, D = rows.shape
    info = plsc.get_sparse_core_info()
    nw = info.num_cores * info.num_subcores
    b_per_w = B // nw
    mesh = plsc.VectorSubcoreMesh(core_axis_name="c", subcore_axis_name="s")

    @functools.partial(
        pl.kernel,
        out_type=jax.ShapeDtypeStruct((T, D), rows.dtype),
        mesh=mesh,
        scratch_types=[
            pltpu.VMEM((b_per_w,), jnp.int32),
            pltpu.VMEM((b_per_w, D), rows.dtype),
            pltpu.SemaphoreType.DMA,
        ],
    )
    def k(rows_hbm, idx_hbm, out_hbm, idx_v, rows_v, sem):
        wid = lax.axis_index("s") * info.num_cores + lax.axis_index("c")
        base = wid * b_per_w
        pltpu.sync_copy(idx_hbm.at[pl.ds(base, b_per_w)], idx_v)
        pltpu.sync_copy(rows_hbm.at[pl.ds(base, b_per_w)], rows_v)
        pltpu.async_copy(rows_v, out_hbm.at[idx_v], sem).wait()

    return k(rows, idx)


def _make_sc_sort_kernels(T):
    info = plsc.get_sparse_core_info()
    NC, NS = info.num_cores, info.num_subcores
    NW = NC * NS
    TPW = T // NW
    NV = TPW // 16
    mesh = plsc.VectorSubcoreMesh(core_axis_name="c", subcore_axis_name="s")

    def _rank_chunks(ids_v, rank_v, hist_v, lane, zeros16):
        for j in range(E // 16):
            hist_v[pl.ds(16 * j, 16)] = zeros16
        for v in range(NV):
            ids = ids_v[pl.ds(16 + 16 * v, 16)]
            rank = zeros16
            cnt = zeros16
            for kk in range(16):
                idk = plsc.load_gather(
                    ids_v, [jnp.full((16,), 16 + 16 * v + kk, jnp.int32)])
                eq = ids == idk
                rank = rank + jnp.where(eq & (lane > kk), 1, 0)
                cnt = cnt + jnp.where(eq, 1, 0)
            pre = plsc.load_gather(hist_v, [ids])
            rank_v[pl.ds(16 * v, 16)] = pre + rank
            plsc.store_scatter(hist_v, [ids], pre + cnt,
                               mask=(cnt - rank) == 1)

    @functools.partial(
        pl.kernel,
        out_type=(jax.ShapeDtypeStruct((T,), jnp.int32),
                  jax.ShapeDtypeStruct((NW, E), jnp.int32)),
        mesh=mesh,
        compiler_params=pltpu.CompilerParams(needs_layout_passes=False),
        scratch_types=[
            pltpu.VMEM((TPW + 16,), jnp.int32),
            pltpu.VMEM((TPW,), jnp.int32),
            pltpu.VMEM((E,), jnp.int32),
        ],
    )
    def k1(ids_hbm, rank_hbm, hist_hbm, ids_v, rank_v, hist_v):
        wid = lax.axis_index("s") * NC + lax.axis_index("c")
        base = wid * TPW
        lane = lax.iota(jnp.int32, 16)
        zeros16 = jnp.zeros((16,), jnp.int32)
        pltpu.sync_copy(ids_hbm.at[pl.ds(base, TPW)], ids_v.at[pl.ds(16, TPW)])
        _rank_chunks(ids_v, rank_v, hist_v, lane, zeros16)
        pltpu.sync_copy(rank_v, rank_hbm.at[pl.ds(base, TPW)])
        pltpu.sync_copy(hist_v, hist_hbm.at[wid])

    @functools.partial(
        pl.kernel,
        out_type=(jax.ShapeDtypeStruct((T,), jnp.int32),
                  jax.ShapeDtypeStruct((80,), jnp.int32),
                  jax.ShapeDtypeStruct((T, H), jnp.float32)),
        mesh=mesh,
        compiler_params=pltpu.CompilerParams(needs_layout_passes=False),
        scratch_types=[
            pltpu.VMEM((TPW,), jnp.int32),
            pltpu.VMEM((TPW,), jnp.int32),
            pltpu.VMEM((TPW,), jnp.int32),
            pltpu.VMEM((NW, E), jnp.int32),
            pltpu.VMEM((E,), jnp.int32),
            pltpu.VMEM((E,), jnp.int32),
            pltpu.VMEM((80,), jnp.int32),
            pltpu.VMEM((16,), jnp.int32),
            pltpu.VMEM((16,), jnp.int32),
            pltpu.VMEM((TPW, H), jnp.float32),
            pltpu.SemaphoreType.DMA,
        ],
    )
    def k2(ids_hbm, rank_hbm, hist_hbm, hid_hbm, pos_hbm, starts_hbm, xs_hbm,
           ids_v, rank_v, pos_v, hist_all_v, base_v, pw_v,
           starts_v, tmp_v, carry_v, rows_v, sem):
        wid = lax.axis_index("s") * NC + lax.axis_index("c")
        base = wid * TPW
        lane = lax.iota(jnp.int32, 16)
        zeros16 = jnp.zeros((16,), jnp.int32)
        pltpu.sync_copy(ids_hbm.at[pl.ds(base, TPW)], ids_v)
        pltpu.sync_copy(rank_hbm.at[pl.ds(base, TPW)], rank_v)
        pltpu.sync_copy(hist_hbm, hist_all_v)
        carry_v[pl.ds(0, 16)] = zeros16
        for j in range(E // 16):
            pw_v[pl.ds(16 * j, 16)] = zeros16
            tot = zeros16
            for w in range(NW):
                h = hist_all_v[w, pl.ds(16 * j, 16)]
                tot = tot + h

                @pl.when(w < wid)
                def _(h=h, j=j):
                    pw_v[pl.ds(16 * j, 16)] = pw_v[pl.ds(16 * j, 16)] + h

            carry = carry_v[pl.ds(0, 16)]
            incc = plsc.cumsum(tot) + carry
            exc = incc - tot
            starts_v[pl.ds(16 * j, 16)] = exc
            base_v[pl.ds(16 * j, 16)] = exc + pw_v[pl.ds(16 * j, 16)]
            tmp_v[pl.ds(0, 16)] = incc
            carry_v[pl.ds(0, 16)] = plsc.load_gather(
                tmp_v, [jnp.full((16,), 15, jnp.int32)])
        starts_v[pl.ds(E, 16)] = jnp.where(lane == 0, T, 0)
        for v in range(NV):
            ids = ids_v[pl.ds(16 * v, 16)]
            b = plsc.load_gather(base_v, [ids])
            pos_v[pl.ds(16 * v, 16)] = b + rank_v[pl.ds(16 * v, 16)]
        pltpu.sync_copy(pos_v, pos_hbm.at[pl.ds(base, TPW)])

        @pl.when(wid == 0)
        def _():
            pltpu.sync_copy(starts_v, starts_hbm)

        pltpu.sync_copy(hid_hbm.at[pl.ds(base, TPW)], rows_v)
        pltpu.async_copy(rows_v, xs_hbm.at[pos_v], sem).wait()

    return k1, k2


def _sc_sort_stage(expert_ids, hidden_flat):
    k1, k2 = _make_sc_sort_kernels(expert_ids.shape[0])
    rank, hist = k1(expert_ids)
    pos, starts_pad, xs = k2(expert_ids, rank, hist, hidden_flat)
    return pos, starts_pad, xs


def _grouped_mlp(starts, xs, gate_w, up_w, down_w):
    T = xs.shape[0]

    def body(starts_ref, xs_ref, gw_ref, uw_ref, dw_ref, out_ref):
        e = pl.program_id(0)
        start = starts_ref[e]
        end = starts_ref[e + 1]
        astart = (start // TILE) * TILE
        n = pl.cdiv(end - astart, TILE)
        gw = gw_ref[0]
        uw = uw_ref[0]
        dw = dw_ref[0]

        def tile_body(i, _):
            off = pl.multiple_of(astart + i * TILE, TILE)
            x = xs_ref[pl.ds(off, TILE), :]
            g = lax.dot_general(x, gw, (((1,), (1,)), ((), ())),
                                preferred_element_type=jnp.float32)
            u = lax.dot_general(x, uw, (((1,), (1,)), ((), ())),
                                preferred_element_type=jnp.float32)
            a = g * jax.nn.sigmoid(g) * u
            y = lax.dot_general(a, dw, (((1,), (1,)), ((), ())),
                                preferred_element_type=jnp.float32)
            rows = off + lax.broadcasted_iota(jnp.int32, (TILE, 1), 0)
            mask = (rows >= start) & (rows < end)
            old = out_ref[pl.ds(off, TILE), :]
            out_ref[pl.ds(off, TILE), :] = jnp.where(mask, y, old)
            return 0

        lax.fori_loop(0, n, tile_body, 0)

    grid_spec = pltpu.PrefetchScalarGridSpec(
        num_scalar_prefetch=1,
        grid=(E,),
        in_specs=[
            pl.BlockSpec((T, H), lambda e, s: (0, 0)),
            pl.BlockSpec((1, F, H), lambda e, s: (e, 0, 0)),
            pl.BlockSpec((1, F, H), lambda e, s: (e, 0, 0)),
            pl.BlockSpec((1, H, F), lambda e, s: (e, 0, 0)),
        ],
        out_specs=pl.BlockSpec((T, H), lambda e, s: (0, 0)),
    )
    return pl.pallas_call(
        body,
        grid_spec=grid_spec,
        out_shape=jax.ShapeDtypeStruct((T, H), jnp.float32),
    )(starts, xs, gate_w, up_w, down_w)


def kernel(hidden_states, routing_weights, selected_experts,
           gate_weight, up_weight, down_weight):
    bsz, seq_len, hidden = hidden_states.shape
    hidden_flat = hidden_states.reshape(-1, hidden)
    T = hidden_flat.shape[0]

    expert_ids = selected_experts.reshape(-1).astype(jnp.int32)
    routing_flat = routing_weights.reshape(-1).astype(jnp.float32)

    pos, starts_pad, xs = _sc_sort_stage(expert_ids, hidden_flat)
    starts = starts_pad[:E + 1]

    ys = _grouped_mlp(starts, xs, gate_weight, up_weight, down_weight)

    out = _sc_gather_scale(ys, pos, routing_flat)
    return out.reshape(bsz, seq_len, hidden)


# speedup vs baseline: 1.0549x; 1.0549x over previous
import functools

import jax
import jax.numpy as jnp
from jax import lax
from jax.experimental import pallas as pl
from jax.experimental.pallas import tpu as pltpu
from jax.experimental.pallas import tpu_sc as plsc

E = 64
H = 1024
F = 768
TILE = 128
EPG = 2


def _sc_gather(table, idx):
    T, D = table.shape
    B = idx.shape[0]
    info = plsc.get_sparse_core_info()
    nw = info.num_cores * info.num_subcores
    b_per_w = B // nw
    mesh = plsc.VectorSubcoreMesh(core_axis_name="c", subcore_axis_name="s")

    @functools.partial(
        pl.kernel,
        out_type=jax.ShapeDtypeStruct((B, D), table.dtype),
        mesh=mesh,
        scratch_types=[
            pltpu.VMEM((b_per_w,), jnp.int32),
            pltpu.VMEM((b_per_w, D), table.dtype),
            pltpu.SemaphoreType.DMA,
        ],
    )
    def k(table_hbm, idx_hbm, out_hbm, idx_v, rows_v, sem):
        wid = lax.axis_index("s") * info.num_cores + lax.axis_index("c")
        base = wid * b_per_w
        pltpu.sync_copy(idx_hbm.at[pl.ds(base, b_per_w)], idx_v)
        pltpu.async_copy(table_hbm.at[idx_v], rows_v, sem).wait()
        pltpu.sync_copy(rows_v, out_hbm.at[pl.ds(base, b_per_w)])

    return k(table, idx)


def _sc_scatter(rows, idx, T):
    B, D = rows.shape
    info = plsc.get_sparse_core_info()
    nw = info.num_cores * info.num_subcores
    b_per_w = B // nw
    mesh = plsc.VectorSubcoreMesh(core_axis_name="c", subcore_axis_name="s")

    @functools.partial(
        pl.kernel,
        out_type=jax.ShapeDtypeStruct((T, D), rows.dtype),
        mesh=mesh,
        scratch_types=[
            pltpu.VMEM((b_per_w,), jnp.int32),
            pltpu.VMEM((b_per_w, D), rows.dtype),
            pltpu.SemaphoreType.DMA,
        ],
    )
    def k(rows_hbm, idx_hbm, out_hbm, idx_v, rows_v, sem):
        wid = lax.axis_index("s") * info.num_cores + lax.axis_index("c")
        base = wid * b_per_w
        pltpu.sync_copy(idx_hbm.at[pl.ds(base, b_per_w)], idx_v)
        pltpu.sync_copy(rows_hbm.at[pl.ds(base, b_per_w)], rows_v)
        pltpu.async_copy(rows_v, out_hbm.at[idx_v], sem).wait()

    return k(rows, idx)


def _make_sc_sort_kernels(T):
    info = plsc.get_sparse_core_info()
    NC, NS = info.num_cores, info.num_subcores
    NW = NC * NS
    TPW = T // NW
    NV = TPW // 16
    mesh = plsc.VectorSubcoreMesh(core_axis_name="c", subcore_axis_name="s")

    def _rank_chunks(ids_v, rank_v, hist_v, lane, zeros16):
        for j in range(E // 16):
            hist_v[pl.ds(16 * j, 16)] = zeros16
        for v in range(NV):
            ids = ids_v[pl.ds(16 + 16 * v, 16)]
            rank = zeros16
            cnt = zeros16
            for kk in range(16):
                idk = plsc.load_gather(
                    ids_v, [jnp.full((16,), 16 + 16 * v + kk, jnp.int32)])
                eq = ids == idk
                rank = rank + jnp.where(eq & (lane > kk), 1, 0)
                cnt = cnt + jnp.where(eq, 1, 0)
            pre = plsc.load_gather(hist_v, [ids])
            rank_v[pl.ds(16 * v, 16)] = pre + rank
            plsc.store_scatter(hist_v, [ids], pre + cnt,
                               mask=(cnt - rank) == 1)

    @functools.partial(
        pl.kernel,
        out_type=(jax.ShapeDtypeStruct((T,), jnp.int32),
                  jax.ShapeDtypeStruct((NW, E), jnp.int32)),
        mesh=mesh,
        compiler_params=pltpu.CompilerParams(needs_layout_passes=False),
        scratch_types=[
            pltpu.VMEM((TPW + 16,), jnp.int32),
            pltpu.VMEM((TPW,), jnp.int32),
            pltpu.VMEM((E,), jnp.int32),
        ],
    )
    def k1(ids_hbm, rank_hbm, hist_hbm, ids_v, rank_v, hist_v):
        wid = lax.axis_index("s") * NC + lax.axis_index("c")
        base = wid * TPW
        lane = lax.iota(jnp.int32, 16)
        zeros16 = jnp.zeros((16,), jnp.int32)
        pltpu.sync_copy(ids_hbm.at[pl.ds(base, TPW)], ids_v.at[pl.ds(16, TPW)])
        _rank_chunks(ids_v, rank_v, hist_v, lane, zeros16)
        pltpu.sync_copy(rank_v, rank_hbm.at[pl.ds(base, TPW)])
        pltpu.sync_copy(hist_v, hist_hbm.at[wid])

    @functools.partial(
        pl.kernel,
        out_type=(jax.ShapeDtypeStruct((T,), jnp.int32),
                  jax.ShapeDtypeStruct((80,), jnp.int32),
                  jax.ShapeDtypeStruct((T, H + 128), jnp.float32)),
        mesh=mesh,
        compiler_params=pltpu.CompilerParams(needs_layout_passes=False),
        scratch_types=[
            pltpu.VMEM((TPW,), jnp.int32),
            pltpu.VMEM((TPW,), jnp.int32),
            pltpu.VMEM((TPW,), jnp.int32),
            pltpu.VMEM((NW, E), jnp.int32),
            pltpu.VMEM((E,), jnp.int32),
            pltpu.VMEM((E,), jnp.int32),
            pltpu.VMEM((80,), jnp.int32),
            pltpu.VMEM((16,), jnp.int32),
            pltpu.VMEM((16,), jnp.int32),
            pltpu.VMEM((TPW, H + 128), jnp.float32),
            pltpu.VMEM((TPW,), jnp.float32),
            pltpu.SemaphoreType.DMA,
        ],
    )
    def k2(ids_hbm, rank_hbm, hist_hbm, hid_hbm, r_hbm, pos_hbm, starts_hbm,
           xs_hbm, ids_v, rank_v, pos_v, hist_all_v, base_v, pw_v,
           starts_v, tmp_v, carry_v, rows_v, r_v, sem):
        wid = lax.axis_index("s") * NC + lax.axis_index("c")
        base = wid * TPW
        lane = lax.iota(jnp.int32, 16)
        zeros16 = jnp.zeros((16,), jnp.int32)
        pltpu.sync_copy(ids_hbm.at[pl.ds(base, TPW)], ids_v)
        pltpu.sync_copy(rank_hbm.at[pl.ds(base, TPW)], rank_v)
        pltpu.sync_copy(hist_hbm, hist_all_v)
        carry_v[pl.ds(0, 16)] = zeros16
        for j in range(E // 16):
            pw_v[pl.ds(16 * j, 16)] = zeros16
            tot = zeros16
            for w in range(NW):
                h = hist_all_v[w, pl.ds(16 * j, 16)]
                tot = tot + h

                @pl.when(w < wid)
                def _(h=h, j=j):
                    pw_v[pl.ds(16 * j, 16)] = pw_v[pl.ds(16 * j, 16)] + h

            carry = carry_v[pl.ds(0, 16)]
            incc = plsc.cumsum(tot) + carry
            exc = incc - tot
            starts_v[pl.ds(16 * j, 16)] = exc
            base_v[pl.ds(16 * j, 16)] = exc + pw_v[pl.ds(16 * j, 16)]
            tmp_v[pl.ds(0, 16)] = incc
            carry_v[pl.ds(0, 16)] = plsc.load_gather(
                tmp_v, [jnp.full((16,), 15, jnp.int32)])
        starts_v[pl.ds(E, 16)] = jnp.where(lane == 0, T, 0)
        for v in range(NV):
            ids = ids_v[pl.ds(16 * v, 16)]
            b = plsc.load_gather(base_v, [ids])
            pos_v[pl.ds(16 * v, 16)] = b + rank_v[pl.ds(16 * v, 16)]
        pltpu.sync_copy(pos_v, pos_hbm.at[pl.ds(base, TPW)])

        @pl.when(wid == 0)
        def _():
            pltpu.sync_copy(starts_v, starts_hbm)

        pltpu.sync_copy(hid_hbm.at[pl.ds(base, TPW)],
                        rows_v.at[:, pl.ds(0, H)])
        pltpu.sync_copy(r_hbm.at[pl.ds(base, TPW)], r_v)

        def row_body(i, _):
            ri = plsc.load_gather(r_v, [jnp.zeros((16,), jnp.int32) + i])
            rows_v[i, pl.ds(H, 16)] = ri
            return 0

        lax.fori_loop(0, TPW, row_body, 0)
        pltpu.async_copy(rows_v, xs_hbm.at[pos_v], sem).wait()

    return k1, k2


def _sc_sort_stage(expert_ids, hidden_flat, routing_flat):
    k1, k2 = _make_sc_sort_kernels(expert_ids.shape[0])
    rank, hist = k1(expert_ids)
    pos, starts_pad, xs = k2(expert_ids, rank, hist, hidden_flat, routing_flat)
    return pos, starts_pad, xs


def _grouped_mlp(starts, xs, gate_w, up_w, down_w):
    T = xs.shape[0]

    def body(starts_ref, xs_ref, gw_ref, uw_ref, dw_ref, out_ref):
        eg = pl.program_id(0)
        for sub in range(EPG):
            e = eg * EPG + sub
            start = starts_ref[e]
            end = starts_ref[e + 1]
            astart = (start // TILE) * TILE
            n = pl.cdiv(end - astart, TILE)
            gw = gw_ref[sub]
            uw = uw_ref[sub]
            dw = dw_ref[sub]

            def tile_body(i, _):
                off = pl.multiple_of(astart + i * TILE, TILE)
                x = xs_ref[pl.ds(off, TILE), :H]
                r = xs_ref[pl.ds(off, TILE), H:H + 1]
                g = lax.dot_general(x, gw, (((1,), (1,)), ((), ())),
                                    preferred_element_type=jnp.float32)
                u = lax.dot_general(x, uw, (((1,), (1,)), ((), ())),
                                    preferred_element_type=jnp.float32)
                a = g * jax.nn.sigmoid(g) * u
                y = lax.dot_general(a, dw, (((1,), (1,)), ((), ())),
                                    preferred_element_type=jnp.float32)
                y = y * r
                rows = off + lax.broadcasted_iota(jnp.int32, (TILE, 1), 0)
                mask = (rows >= start) & (rows < end)
                old = out_ref[pl.ds(off, TILE), :]
                out_ref[pl.ds(off, TILE), :] = jnp.where(mask, y, old)
                return 0

            lax.fori_loop(0, n, tile_body, 0)

    grid_spec = pltpu.PrefetchScalarGridSpec(
        num_scalar_prefetch=1,
        grid=(E // EPG,),
        in_specs=[
            pl.BlockSpec((T, H + 128), lambda e, s: (0, 0)),
            pl.BlockSpec((EPG, F, H), lambda e, s: (e, 0, 0)),
            pl.BlockSpec((EPG, F, H), lambda e, s: (e, 0, 0)),
            pl.BlockSpec((EPG, H, F), lambda e, s: (e, 0, 0)),
        ],
        out_specs=pl.BlockSpec((T, H), lambda e, s: (0, 0)),
    )
    return pl.pallas_call(
        body,
        grid_spec=grid_spec,
        out_shape=jax.ShapeDtypeStruct((T, H), jnp.float32),
        compiler_params=pltpu.CompilerParams(
            vmem_limit_bytes=100 * 1024 * 1024),
    )(starts, xs, gate_w, up_w, down_w)


def kernel(hidden_states, routing_weights, selected_experts,
           gate_weight, up_weight, down_weight):
    bsz, seq_len, hidden = hidden_states.shape
    hidden_flat = hidden_states.reshape(-1, hidden)
    T = hidden_flat.shape[0]

    expert_ids = selected_experts.reshape(-1).astype(jnp.int32)
    routing_flat = routing_weights.reshape(-1).astype(jnp.float32)

    pos, starts_pad, xs = _sc_sort_stage(expert_ids, hidden_flat, routing_flat)
    starts = starts_pad[:E + 1]

    ys = _grouped_mlp(starts, xs, gate_weight, up_weight, down_weight)

    out = _sc_gather(ys, pos)
    return out.reshape(bsz, seq_len, hidden)


# speedup vs baseline: 1.0551x; 1.0002x over previous
import functools

import jax
import jax.numpy as jnp
from jax import lax
from jax.experimental import pallas as pl
from jax.experimental.pallas import tpu as pltpu
from jax.experimental.pallas import tpu_sc as plsc

E = 64
H = 1024
F = 768
TILE = 128
EPG = 2


def _sc_gather(table, idx):
    T, D = table.shape
    B = idx.shape[0]
    info = plsc.get_sparse_core_info()
    nw = info.num_cores * info.num_subcores
    b_per_w = B // nw
    mesh = plsc.VectorSubcoreMesh(core_axis_name="c", subcore_axis_name="s")

    @functools.partial(
        pl.kernel,
        out_type=jax.ShapeDtypeStruct((B, D), table.dtype),
        mesh=mesh,
        scratch_types=[
            pltpu.VMEM((b_per_w,), jnp.int32),
            pltpu.VMEM((b_per_w, D), table.dtype),
            pltpu.SemaphoreType.DMA,
        ],
    )
    def k(table_hbm, idx_hbm, out_hbm, idx_v, rows_v, sem):
        wid = lax.axis_index("s") * info.num_cores + lax.axis_index("c")
        base = wid * b_per_w
        pltpu.sync_copy(idx_hbm.at[pl.ds(base, b_per_w)], idx_v)
        pltpu.async_copy(table_hbm.at[idx_v], rows_v, sem).wait()
        pltpu.sync_copy(rows_v, out_hbm.at[pl.ds(base, b_per_w)])

    return k(table, idx)


def _make_sc_sort_kernels(T):
    info = plsc.get_sparse_core_info()
    NC, NS = info.num_cores, info.num_subcores
    NW = NC * NS
    TPW = T // NW
    NV = TPW // 16
    mesh = plsc.VectorSubcoreMesh(core_axis_name="c", subcore_axis_name="s")

    def _rank_chunks(ids_v, rank_v, hist_v, lane, zeros16):
        for j in range(E // 16):
            hist_v[pl.ds(16 * j, 16)] = zeros16
        for v in range(NV):
            ids = ids_v[pl.ds(16 + 16 * v, 16)]
            rank = zeros16
            cnt = zeros16
            for kk in range(16):
                idk = plsc.load_gather(
                    ids_v, [jnp.full((16,), 16 + 16 * v + kk, jnp.int32)])
                eq = ids == idk
                rank = rank + jnp.where(eq & (lane > kk), 1, 0)
                cnt = cnt + jnp.where(eq, 1, 0)
            pre = plsc.load_gather(hist_v, [ids])
            rank_v[pl.ds(16 * v, 16)] = pre + rank
            plsc.store_scatter(hist_v, [ids], pre + cnt,
                               mask=(cnt - rank) == 1)

    @functools.partial(
        pl.kernel,
        out_type=(jax.ShapeDtypeStruct((T,), jnp.int32),
                  jax.ShapeDtypeStruct((NW, E), jnp.int32)),
        mesh=mesh,
        compiler_params=pltpu.CompilerParams(needs_layout_passes=False),
        scratch_types=[
            pltpu.VMEM((TPW + 16,), jnp.int32),
            pltpu.VMEM((TPW,), jnp.int32),
            pltpu.VMEM((E,), jnp.int32),
        ],
    )
    def k1(ids_hbm, rank_hbm, hist_hbm, ids_v, rank_v, hist_v):
        wid = lax.axis_index("s") * NC + lax.axis_index("c")
        base = wid * TPW
        lane = lax.iota(jnp.int32, 16)
        zeros16 = jnp.zeros((16,), jnp.int32)
        pltpu.sync_copy(ids_hbm.at[pl.ds(base, TPW)], ids_v.at[pl.ds(16, TPW)])
        _rank_chunks(ids_v, rank_v, hist_v, lane, zeros16)
        pltpu.sync_copy(rank_v, rank_hbm.at[pl.ds(base, TPW)])
        pltpu.sync_copy(hist_v, hist_hbm.at[wid])

    @functools.partial(
        pl.kernel,
        out_type=(jax.ShapeDtypeStruct((T,), jnp.int32),
                  jax.ShapeDtypeStruct((80,), jnp.int32),
                  jax.ShapeDtypeStruct((T, H + 128), jnp.float32)),
        mesh=mesh,
        compiler_params=pltpu.CompilerParams(needs_layout_passes=False),
        scratch_types=[
            pltpu.VMEM((TPW,), jnp.int32),
            pltpu.VMEM((TPW,), jnp.int32),
            pltpu.VMEM((TPW,), jnp.int32),
            pltpu.VMEM((NW, E), jnp.int32),
            pltpu.VMEM((E,), jnp.int32),
            pltpu.VMEM((E,), jnp.int32),
            pltpu.VMEM((80,), jnp.int32),
            pltpu.VMEM((16,), jnp.int32),
            pltpu.VMEM((16,), jnp.int32),
            pltpu.VMEM((TPW, H + 128), jnp.float32),
            pltpu.VMEM((TPW,), jnp.float32),
            pltpu.SemaphoreType.DMA,
        ],
    )
    def k2(ids_hbm, rank_hbm, hist_hbm, hid_hbm, r_hbm, pos_hbm, starts_hbm,
           xs_hbm, ids_v, rank_v, pos_v, hist_all_v, base_v, pw_v,
           starts_v, tmp_v, carry_v, rows_v, r_v, sem):
        wid = lax.axis_index("s") * NC + lax.axis_index("c")
        base = wid * TPW
        lane = lax.iota(jnp.int32, 16)
        zeros16 = jnp.zeros((16,), jnp.int32)
        pltpu.sync_copy(ids_hbm.at[pl.ds(base, TPW)], ids_v)
        pltpu.sync_copy(rank_hbm.at[pl.ds(base, TPW)], rank_v)
        pltpu.sync_copy(hist_hbm, hist_all_v)
        carry_v[pl.ds(0, 16)] = zeros16
        for j in range(E // 16):
            pw_v[pl.ds(16 * j, 16)] = zeros16
            tot = zeros16
            for w in range(NW):
                h = hist_all_v[w, pl.ds(16 * j, 16)]
                tot = tot + h

                @pl.when(w < wid)
                def _(h=h, j=j):
                    pw_v[pl.ds(16 * j, 16)] = pw_v[pl.ds(16 * j, 16)] + h

            carry = carry_v[pl.ds(0, 16)]
            incc = plsc.cumsum(tot) + carry
            exc = incc - tot
            starts_v[pl.ds(16 * j, 16)] = exc
            base_v[pl.ds(16 * j, 16)] = exc + pw_v[pl.ds(16 * j, 16)]
            tmp_v[pl.ds(0, 16)] = incc
            carry_v[pl.ds(0, 16)] = plsc.load_gather(
                tmp_v, [jnp.full((16,), 15, jnp.int32)])
        starts_v[pl.ds(E, 16)] = jnp.where(lane == 0, T, 0)
        for v in range(NV):
            ids = ids_v[pl.ds(16 * v, 16)]
            b = plsc.load_gather(base_v, [ids])
            pos_v[pl.ds(16 * v, 16)] = b + rank_v[pl.ds(16 * v, 16)]
        pltpu.sync_copy(pos_v, pos_hbm.at[pl.ds(base, TPW)])

        @pl.when(wid == 0)
        def _():
            pltpu.sync_copy(starts_v, starts_hbm)

        pltpu.sync_copy(hid_hbm.at[pl.ds(base, TPW)],
                        rows_v.at[:, pl.ds(0, H)])
        pltpu.sync_copy(r_hbm.at[pl.ds(base, TPW)], r_v)

        def row_body(i, _):
            ri = plsc.load_gather(r_v, [jnp.zeros((16,), jnp.int32) + i])
            rows_v[i, pl.ds(H, 16)] = ri
            return 0

        lax.fori_loop(0, TPW, row_body, 0)
        pltpu.async_copy(rows_v, xs_hbm.at[pos_v], sem).wait()

    return k1, k2


def _sc_sort_stage(expert_ids, hidden_flat, routing_flat):
    k1, k2 = _make_sc_sort_kernels(expert_ids.shape[0])
    rank, hist = k1(expert_ids)
    pos, starts_pad, xs = k2(expert_ids, rank, hist, hidden_flat, routing_flat)
    return pos, starts_pad, xs


def _grouped_mlp(starts, xs, gate_w, up_w, down_w):
    T = xs.shape[0]

    def body(starts_ref, xs_ref, gw_ref, uw_ref, dw_ref, out_ref):
        eg = pl.program_id(0)
        for sub in range(EPG):
            e = eg * EPG + sub
            start = starts_ref[e]
            end = starts_ref[e + 1]
            astart = (start // TILE) * TILE
            n = pl.cdiv(end - astart, TILE)
            gw = gw_ref[sub]
            uw = uw_ref[sub]
            dw = dw_ref[sub]

            def tile_body(i, _):
                off = pl.multiple_of(astart + i * TILE, TILE)
                x = xs_ref[pl.ds(off, TILE), :H]
                r = xs_ref[pl.ds(off, TILE), H:H + 1]
                g = lax.dot_general(x, gw, (((1,), (1,)), ((), ())),
                                    preferred_element_type=jnp.float32)
                u = lax.dot_general(x, uw, (((1,), (1,)), ((), ())),
                                    preferred_element_type=jnp.float32)
                a = g * jax.nn.sigmoid(g) * u
                y = lax.dot_general(a, dw, (((1,), (1,)), ((), ())),
                                    preferred_element_type=jnp.float32)
                y = y * r
                rows = off + lax.broadcasted_iota(jnp.int32, (TILE, 1), 0)
                mask = (rows >= start) & (rows < end)
                old = out_ref[pl.ds(off, TILE), :]
                out_ref[pl.ds(off, TILE), :] = jnp.where(mask, y, old)
                return 0

            lax.fori_loop(0, n, tile_body, 0)

    grid_spec = pltpu.PrefetchScalarGridSpec(
        num_scalar_prefetch=1,
        grid=(E // EPG,),
        in_specs=[
            pl.BlockSpec((T, H + 128), lambda e, s: (0, 0)),
            pl.BlockSpec((EPG, F, H), lambda e, s: (e, 0, 0)),
            pl.BlockSpec((EPG, F, H), lambda e, s: (e, 0, 0)),
            pl.BlockSpec((EPG, H, F), lambda e, s: (e, 0, 0)),
        ],
        out_specs=pl.BlockSpec((T, H), lambda e, s: (0, 0)),
    )
    return pl.pallas_call(
        body,
        grid_spec=grid_spec,
        out_shape=jax.ShapeDtypeStruct((T, H), jnp.float32),
        compiler_params=pltpu.CompilerParams(
            vmem_limit_bytes=100 * 1024 * 1024),
    )(starts, xs, gate_w, up_w, down_w)


def kernel(hidden_states, routing_weights, selected_experts,
           gate_weight, up_weight, down_weight):
    bsz, seq_len, hidden = hidden_states.shape
    hidden_flat = hidden_states.reshape(-1, hidden)
    T = hidden_flat.shape[0]

    expert_ids = selected_experts.reshape(-1).astype(jnp.int32)
    routing_flat = routing_weights.reshape(-1).astype(jnp.float32)

    pos, starts_pad, xs = _sc_sort_stage(expert_ids, hidden_flat, routing_flat)
    starts = starts_pad[:E + 1]

    ys = _grouped_mlp(starts, xs, gate_weight, up_weight, down_weight)

    out = _sc_gather(ys, pos)
    return out.reshape(bsz, seq_len, hidden)
